# baseline (speedup 1.0000x reference)
import functools

import jax
import jax.numpy as jnp
from jax import lax
from jax.experimental import pallas as pl
from jax.experimental.pallas import tpu as pltpu
from jax.experimental.pallas import tpu_sc as plsc

EMB_DIM = 64
BATCH = 16384
CTX = 10
NEG = 5

NC = 2
NS = 16
NW = NC * NS
B_PER_W = BATCH // NW
E = 32
NCHUNK = B_PER_W // E


def _sc_body(u_table, w_table, u_idx_h, w_idx_h, n_idx_h, out_h,
             u_idx_v, w_idx_v, n_idx_v, u_rows, w_rows, n_rows, scores_v, sem):
    wid = lax.axis_index("s") * NC + lax.axis_index("c")

    pltpu.sync_copy(u_idx_h.at[wid], u_idx_v)
    pltpu.sync_copy(w_idx_h.at[wid], w_idx_v)
    pltpu.sync_copy(n_idx_h.at[wid], n_idx_v)

    def chunk_body(j, carry):
        descs = []
        for r in range(CTX):
            descs.append(pltpu.async_copy(
                u_table.at[u_idx_v.at[j, r]], u_rows.at[pl.ds(r * E, E)], sem))
        descs.append(pltpu.async_copy(
            w_table.at[w_idx_v.at[j]], w_rows, sem))
        for k in range(NEG):
            descs.append(pltpu.async_copy(
                w_table.at[n_idx_v.at[j, k]], n_rows.at[pl.ds(k * E, E)], sem))
        for d in descs:
            d.wait()

        def ex_body(e, carry2):
            h = []
            for q in range(EMB_DIM // 16):
                acc = u_rows[e * CTX, pl.ds(q * 16, 16)]
                for c in range(1, CTX):
                    acc = acc + u_rows[e * CTX + c, pl.ds(q * 16, 16)]
                h.append(acc)
            col = j * E + e
            p = h[0] * w_rows[e, pl.ds(0, 16)]
            for q in range(1, EMB_DIM // 16):
                p = p + h[q] * w_rows[e, pl.ds(q * 16, 16)]
            scores_v[0, col, :] = p
            for k in range(NEG):
                p = h[0] * n_rows[e * NEG + k, pl.ds(0, 16)]
                for q in range(1, EMB_DIM // 16):
                    p = p + h[q] * n_rows[e * NEG + k, pl.ds(q * 16, 16)]
                scores_v[1 + k, col, :] = p
            return carry2
        lax.fori_loop(0, E, ex_body, 0)
        return carry
    lax.fori_loop(0, NCHUNK, chunk_body, 0)

    pltpu.sync_copy(scores_v, out_h.at[wid])


@functools.cache
def _sc_scores():
    mesh = plsc.VectorSubcoreMesh(
        core_axis_name="c", subcore_axis_name="s",
        num_cores=NC, num_subcores=NS)
    return pl.kernel(
        _sc_body,
        out_type=jax.ShapeDtypeStruct((NW, 1 + NEG, B_PER_W, 16), jnp.float32),
        mesh=mesh,
        compiler_params=pltpu.CompilerParams(use_tc_tiling_on_sc=False),
        scratch_types=[
            pltpu.VMEM((NCHUNK, CTX, E), jnp.int32),
            pltpu.VMEM((NCHUNK, E), jnp.int32),
            pltpu.VMEM((NCHUNK, NEG, E), jnp.int32),
            pltpu.VMEM((CTX * E, EMB_DIM), jnp.float32),
            pltpu.VMEM((E, EMB_DIM), jnp.float32),
            pltpu.VMEM((NEG * E, EMB_DIM), jnp.float32),
            pltpu.VMEM((1 + NEG, B_PER_W, 16), jnp.float32),
            pltpu.SemaphoreType.DMA,
        ],
    )


def _loss_body(s_ref, o_ref):
    s = s_ref[...]
    sc = jnp.sum(s, axis=-1)
    pos = sc[:, 0, :]
    neg = sc[:, 1:, :]

    def logsig(x):
        return jnp.minimum(x, 0.0) - jnp.log1p(jnp.exp(-jnp.abs(x)))

    o_ref[0, 0] = -(jnp.sum(logsig(pos)) + jnp.sum(logsig(-neg)))


def kernel(u_table, w_table, pos_u, pos_w, neg_w):
    u_idx = pos_u.reshape(NW, NCHUNK, CTX, E).astype(jnp.int32)
    w_idx = pos_w.reshape(NW, NCHUNK, E).astype(jnp.int32)
    n_idx = neg_w.reshape(NW, NCHUNK, NEG, E).astype(jnp.int32)

    scores = _sc_scores()(u_table, w_table, u_idx, w_idx, n_idx)

    loss = pl.pallas_call(
        _loss_body,
        out_shape=jax.ShapeDtypeStruct((1, 1), jnp.float32),
        out_specs=pl.BlockSpec(memory_space=pltpu.SMEM),
    )(scores)
    return loss[0, 0]


# speedup vs baseline: 1.0354x; 1.0354x over previous
import functools

import jax
import jax.numpy as jnp
from jax import lax
from jax.experimental import pallas as pl
from jax.experimental.pallas import tpu as pltpu
from jax.experimental.pallas import tpu_sc as plsc

EMB_DIM = 64
PAD_DIM = 128
BATCH = 16384
CTX = 10
NEG = 5

NC = 2
NS = 16
NW = NC * NS
B_PER_W = BATCH // NW
E = 32
NCHUNK = B_PER_W // E
NSC = B_PER_W * 16


def _sc_body(u_table, w_table, u_idx_h, w_idx_h, n_idx_h, out_h,
             u_idx_v, w_idx_v, n_idx_v, u_rows, w_rows, n_rows, scores_v, sem):
    wid = lax.axis_index("s") * NC + lax.axis_index("c")

    pltpu.sync_copy(u_idx_h.at[wid], u_idx_v)
    pltpu.sync_copy(w_idx_h.at[wid], w_idx_v)
    pltpu.sync_copy(n_idx_h.at[wid], n_idx_v)

    def chunk_body(j, carry):
        descs = []
        for r in range(CTX):
            descs.append(pltpu.async_copy(
                u_table.at[u_idx_v.at[pl.ds(j * CTX * E + r * E, E)]],
                u_rows.at[pl.ds(r * E, E)], sem))
        descs.append(pltpu.async_copy(
            w_table.at[w_idx_v.at[pl.ds(j * E, E)]], w_rows, sem))
        for k in range(NEG):
            descs.append(pltpu.async_copy(
                w_table.at[n_idx_v.at[pl.ds(j * NEG * E + k * E, E)]],
                n_rows.at[pl.ds(k * E, E)], sem))
        for d in descs:
            d.wait()

        def ex_body(e, carry2):
            h = []
            for q in range(EMB_DIM // 16):
                acc = u_rows[e * CTX, pl.ds(q * 16, 16)]
                for c in range(1, CTX):
                    acc = acc + u_rows[e * CTX + c, pl.ds(q * 16, 16)]
                h.append(acc)
            col = e * 16
            p = h[0] * w_rows[e, pl.ds(0, 16)]
            for q in range(1, EMB_DIM // 16):
                p = p + h[q] * w_rows[e, pl.ds(q * 16, 16)]
            scores_v[0, pl.ds(col, 16)] = p
            for k in range(NEG):
                p = h[0] * n_rows[e * NEG + k, pl.ds(0, 16)]
                for q in range(1, EMB_DIM // 16):
                    p = p + h[q] * n_rows[e * NEG + k, pl.ds(q * 16, 16)]
                scores_v[1 + k, pl.ds(col, 16)] = p
            return carry2
        lax.fori_loop(0, E, ex_body, 0)
        pltpu.sync_copy(scores_v, out_h.at[wid, j])
        return carry
    lax.fori_loop(0, NCHUNK, chunk_body, 0)


@functools.cache
def _sc_scores():
    mesh = plsc.VectorSubcoreMesh(
        core_axis_name="c", subcore_axis_name="s",
        num_cores=NC, num_subcores=NS)
    return pl.kernel(
        _sc_body,
        out_type=jax.ShapeDtypeStruct((NW, NCHUNK, 1 + NEG, E * 16), jnp.float32),
        mesh=mesh,
        scratch_types=[
            pltpu.VMEM((NCHUNK * CTX * E,), jnp.int32),
            pltpu.VMEM((NCHUNK * E,), jnp.int32),
            pltpu.VMEM((NCHUNK * NEG * E,), jnp.int32),
            pltpu.VMEM((CTX * E, PAD_DIM), jnp.float32),
            pltpu.VMEM((E, PAD_DIM), jnp.float32),
            pltpu.VMEM((NEG * E, PAD_DIM), jnp.float32),
            pltpu.VMEM((1 + NEG, E * 16), jnp.float32),
            pltpu.SemaphoreType.DMA,
        ],
    )


def _loss_body(s_ref, o_ref):
    s = s_ref[...]
    sc = jnp.sum(s, axis=-1)
    pos = sc[:, 0, :]
    neg = sc[:, 1:, :]

    def logsig(x):
        return jnp.minimum(x, 0.0) - jnp.log1p(jnp.exp(-jnp.abs(x)))

    o_ref[0, 0] = -(jnp.sum(logsig(pos)) + jnp.sum(logsig(-neg)))


def kernel(u_table, w_table, pos_u, pos_w, neg_w):
    u_pad = jnp.pad(u_table, ((0, 0), (0, PAD_DIM - EMB_DIM)))
    w_pad = jnp.pad(w_table, ((0, 0), (0, PAD_DIM - EMB_DIM)))

    u_idx = pos_u.reshape(NW, NCHUNK * CTX * E).astype(jnp.int32)
    w_idx = pos_w.reshape(NW, NCHUNK * E).astype(jnp.int32)
    n_idx = neg_w.reshape(NW, NCHUNK * NEG * E).astype(jnp.int32)

    scores = _sc_scores()(u_pad, w_pad, u_idx, w_idx, n_idx)

    loss = pl.pallas_call(
        _loss_body,
        out_shape=jax.ShapeDtypeStruct((1, 1), jnp.float32),
        out_specs=pl.BlockSpec(memory_space=pltpu.SMEM),
    )(scores.reshape(NW * NCHUNK, 1 + NEG, E, 16))
    return loss[0, 0]


# speedup vs baseline: 1.0973x; 1.0598x over previous
import functools

import jax
import jax.numpy as jnp
from jax import lax
from jax.experimental import pallas as pl
from jax.experimental.pallas import tpu as pltpu
from jax.experimental.pallas import tpu_sc as plsc

EMB_DIM = 64
PAD_DIM = 128
BATCH = 16384
CTX = 10
NEG = 5

NC = 2
NS = 16
NW = NC * NS
B_PER_W = BATCH // NW
E = 32
NCHUNK = B_PER_W // E
NSC = B_PER_W * 16


def _sc_body(u_table, w_table, u_idx_h, w_idx_h, n_idx_h, out_h,
             u_idx_v, w_idx_v, n_idx_v, u_rows, w_rows, n_rows, scores_v, sem):
    wid = lax.axis_index("s") * NC + lax.axis_index("c")

    pltpu.sync_copy(u_idx_h.at[wid], u_idx_v)
    pltpu.sync_copy(w_idx_h.at[wid], w_idx_v)
    pltpu.sync_copy(n_idx_h.at[wid], n_idx_v)

    def chunk_body(j, carry):
        descs = []
        for r in range(CTX):
            descs.append(pltpu.async_copy(
                u_table.at[u_idx_v.at[pl.ds(j * CTX * E + r * E, E)]],
                u_rows.at[pl.ds(r * E, E)], sem))
        descs.append(pltpu.async_copy(
            w_table.at[w_idx_v.at[pl.ds(j * E, E)]], w_rows, sem))
        for k in range(NEG):
            descs.append(pltpu.async_copy(
                w_table.at[n_idx_v.at[pl.ds(j * NEG * E + k * E, E)]],
                n_rows.at[pl.ds(k * E, E)], sem))
        for d in descs:
            d.wait()

        def ex_body(e, carry2):
            h = []
            for q in range(EMB_DIM // 16):
                acc = u_rows[e * CTX, pl.ds(q * 16, 16)]
                for c in range(1, CTX):
                    acc = acc + u_rows[e * CTX + c, pl.ds(q * 16, 16)]
                h.append(acc)
            col = e * 16
            p = h[0] * w_rows[e, pl.ds(0, 16)]
            for q in range(1, EMB_DIM // 16):
                p = p + h[q] * w_rows[e, pl.ds(q * 16, 16)]
            scores_v[0, pl.ds(col, 16)] = p
            for k in range(NEG):
                p = h[0] * n_rows[e * NEG + k, pl.ds(0, 16)]
                for q in range(1, EMB_DIM // 16):
                    p = p + h[q] * n_rows[e * NEG + k, pl.ds(q * 16, 16)]
                scores_v[1 + k, pl.ds(col, 16)] = p
            return carry2
        lax.fori_loop(0, E, ex_body, 0)
        pltpu.sync_copy(scores_v, out_h.at[wid, j])
        return carry
    lax.fori_loop(0, NCHUNK, chunk_body, 0)


@functools.cache
def _sc_scores():
    mesh = plsc.VectorSubcoreMesh(
        core_axis_name="c", subcore_axis_name="s",
        num_cores=NC, num_subcores=NS)
    return pl.kernel(
        _sc_body,
        out_type=jax.ShapeDtypeStruct((NW, NCHUNK, 1 + NEG, E * 16), jnp.float32),
        mesh=mesh,
        scratch_types=[
            pltpu.VMEM((NCHUNK * CTX * E,), jnp.int32),
            pltpu.VMEM((NCHUNK * E,), jnp.int32),
            pltpu.VMEM((NCHUNK * NEG * E,), jnp.int32),
            pltpu.VMEM((CTX * E, PAD_DIM), jnp.float32),
            pltpu.VMEM((E, PAD_DIM), jnp.float32),
            pltpu.VMEM((NEG * E, PAD_DIM), jnp.float32),
            pltpu.VMEM((1 + NEG, E * 16), jnp.float32),
            pltpu.SemaphoreType.DMA,
        ],
    )


_TR_B = 2048


def _tr_body(x_ref, o_ref):
    o_ref[:, 0:EMB_DIM] = jnp.transpose(x_ref[...], (1, 0))


@functools.cache
def _to_padded_rows(n_rows):
    grid = -(-n_rows // _TR_B)
    return pl.pallas_call(
        _tr_body,
        grid=(grid,),
        in_specs=[pl.BlockSpec((EMB_DIM, _TR_B), lambda i: (0, i))],
        out_specs=pl.BlockSpec((_TR_B, PAD_DIM), lambda i: (i, 0)),
        out_shape=jax.ShapeDtypeStruct((n_rows, PAD_DIM), jnp.float32),
    )


def _loss_body(s_ref, o_ref):
    s = s_ref[...]
    sc = jnp.sum(s, axis=-1)
    pos = sc[:, 0, :]
    neg = sc[:, 1:, :]

    def logsig(x):
        return jnp.minimum(x, 0.0) - jnp.log1p(jnp.exp(-jnp.abs(x)))

    o_ref[0, 0] = -(jnp.sum(logsig(pos)) + jnp.sum(logsig(-neg)))


def kernel(u_table, w_table, pos_u, pos_w, neg_w):
    n_rows = u_table.shape[0]
    tr = _to_padded_rows(n_rows)
    u_pad = tr(u_table.T)
    w_pad = tr(w_table.T)

    u_idx = pos_u.reshape(NW, NCHUNK * CTX * E).astype(jnp.int32)
    w_idx = pos_w.reshape(NW, NCHUNK * E).astype(jnp.int32)
    n_idx = neg_w.reshape(NW, NCHUNK * NEG * E).astype(jnp.int32)

    scores = _sc_scores()(u_pad, w_pad, u_idx, w_idx, n_idx)

    loss = pl.pallas_call(
        _loss_body,
        out_shape=jax.ShapeDtypeStruct((1, 1), jnp.float32),
        out_specs=pl.BlockSpec(memory_space=pltpu.SMEM),
    )(scores.reshape(NW * NCHUNK, 1 + NEG, E, 16))
    return loss[0, 0]
